# baseline (speedup 1.0000x reference)
import functools

import jax
import jax.numpy as jnp
from jax import lax
from jax.experimental import pallas as pl
from jax.experimental.pallas import tpu as pltpu
from jax.experimental.pallas import tpu_sc as plsc

_BATCH = 16384
_SEQ = 200
_D = 64
_B = _BATCH * _SEQ

_NC = 2
_NS = 16
_NW = _NC * _NS

_B_PER_W = _B // _NW
_CHUNK = 400
_NCH = _B_PER_W // _CHUNK
_NBUF = 4

_mesh = plsc.VectorSubcoreMesh(
    core_axis_name="c", subcore_axis_name="s", num_cores=_NC, num_subcores=_NS
)


@functools.partial(
    pl.kernel,
    out_type=jax.ShapeDtypeStruct((_B, _D), jnp.float32),
    mesh=_mesh,
    scratch_types=[
        [pltpu.VMEM((_CHUNK,), jnp.int32) for _ in range(_NBUF)],
        [pltpu.VMEM((_CHUNK, _D), jnp.float32) for _ in range(_NBUF)],
        [pltpu.SemaphoreType.DMA for _ in range(_NBUF)],
        [pltpu.SemaphoreType.DMA for _ in range(_NBUF)],
        [pltpu.SemaphoreType.DMA for _ in range(_NBUF)],
    ],
    compiler_params=pltpu.CompilerParams(use_tc_tiling_on_sc=False),
)
def _embed_sc(idx_hbm, table_hbm, out_hbm, idx_v, rows_v, sem_i, sem_g, sem_o):
    wid = lax.axis_index("s") * _NC + lax.axis_index("c")
    base = wid * _B_PER_W

    def start_idx(g, slot):
        off = base + g * _CHUNK
        pltpu.async_copy(idx_hbm.at[pl.ds(off, _CHUNK)], idx_v[slot], sem_i[slot])

    def wait_idx(slot):
        pltpu.make_async_copy(
            idx_hbm.at[pl.ds(0, _CHUNK)], idx_v[slot], sem_i[slot]
        ).wait()

    def start_gather(slot):
        pltpu.async_copy(table_hbm.at[idx_v[slot]], rows_v[slot], sem_g[slot])

    def wait_gather(slot):
        pltpu.make_async_copy(
            table_hbm.at[idx_v[slot]], rows_v[slot], sem_g[slot]
        ).wait()

    def start_out(g, slot):
        off = base + g * _CHUNK
        pltpu.async_copy(rows_v[slot], out_hbm.at[pl.ds(off, _CHUNK)], sem_o[slot])

    def wait_out(slot):
        pltpu.make_async_copy(
            rows_v[slot], out_hbm.at[pl.ds(0, _CHUNK)], sem_o[slot]
        ).wait()


    start_idx(0, 0)
    start_idx(1, 1)
    wait_idx(0)
    start_gather(0)
    for g in (0, 1):
        start_idx(g + 2, g + 2)
        wait_idx(g + 1)
        start_gather(g + 1)
        wait_gather(g)
        start_out(g, g)

    def body(gg, carry):
        for j in range(_NBUF):
            g = 2 + gg * _NBUF + j
            b = (2 + j) % _NBUF
            wait_out(j)
            start_idx(g + 2, j)
            wait_idx((3 + j) % _NBUF)
            start_gather((3 + j) % _NBUF)
            wait_gather(b)
            start_out(g, b)
        return carry

    lax.fori_loop(0, (_NCH - 4) // _NBUF, body, 0)

    g = _NCH - 2
    wait_idx((g + 1) % _NBUF)
    start_gather((g + 1) % _NBUF)
    wait_gather(g % _NBUF)
    start_out(g, g % _NBUF)
    g = _NCH - 1
    wait_gather(g % _NBUF)
    start_out(g, g % _NBUF)
    for j in range(_NBUF):
        wait_out(j)


def kernel(idx, table):
    out = _embed_sc(idx.reshape(_B), table)
    return out.reshape(_BATCH, _SEQ, _D)
